# baseline (speedup 1.0000x reference)
import functools

import jax
import jax.numpy as jnp
from jax import lax
from jax.experimental import pallas as pl
from jax.experimental.pallas import tpu as pltpu
from jax.experimental.pallas import tpu_sc as plsc

N = 10000
D = 128
E = 320000

NC = 2
NS = 16
NW = NC * NS

PIECES = [(0, 32000, 40), (32000, 64000, 80), (96000, 96000, 120),
          (192000, 128000, 80)]
P = len(PIECES)
KBUF = 5

PER_W_S = E // NW
CH_S = 40
NCHUNK_S = PER_W_S // CH_S
NGROUP_S = NCHUNK_S // KBUF

ROWS_PER_SUB = 624
ROWS_TAIL = N - NS * ROWS_PER_SUB

_mesh = plsc.VectorSubcoreMesh(core_axis_name="c", subcore_axis_name="s")


def _make_gather(poff, ep, ch):
    per_w = ep // NW
    ngroup = per_w // (ch * KBUF)
    assert ngroup * ch * KBUF == per_w and ch % 8 == 0 and per_w % 8 == 0

    @functools.partial(
        pl.kernel,
        out_type=jax.ShapeDtypeStruct((ep, D), jnp.float32),
        mesh=_mesh,
        scratch_types=(
            [pltpu.VMEM((per_w,), jnp.int32),
             pltpu.VMEM((KBUF, ch, D), jnp.float32)]
            + [pltpu.SemaphoreType.DMA] * (2 * KBUF)
        ),
    )
    def _sc_gather(m_hbm, src_hbm, out_hbm, idx_all, rows, *sems):
        gsem = sems[:KBUF]
        wsem = sems[KBUF:]
        wid = lax.axis_index("s") * NC + lax.axis_index("c")
        base = wid * per_w
        pltpu.sync_copy(src_hbm.at[pl.ds(poff + base, per_w)], idx_all)

        def group(gi, carry):
            rel = gi * (KBUF * ch)
            descs = []
            for b in range(KBUF):
                @pl.when(gi > 0)
                def _(b=b):
                    pltpu.make_async_copy(
                        rows.at[b], out_hbm.at[pl.ds(base, ch)],
                        wsem[b]).wait()

                d = pltpu.async_copy(
                    m_hbm.at[idx_all.at[pl.ds(rel + b * ch, ch)]],
                    rows.at[b], gsem[b])
                descs.append(d)
            for b in range(KBUF):
                descs[b].wait()
                pltpu.async_copy(rows.at[b],
                                 out_hbm.at[pl.ds(base + rel + b * ch, ch)],
                                 wsem[b])
            return carry

        lax.fori_loop(0, ngroup, group, 0)
        for b in range(KBUF):
            pltpu.make_async_copy(
                rows.at[b], out_hbm.at[pl.ds(base, ch)], wsem[b]).wait()

    return _sc_gather


_sc_gathers = [_make_gather(*pc) for pc in PIECES]


@functools.partial(
    pl.kernel,
    out_type=(
        jax.ShapeDtypeStruct((N, D), jnp.float32),
        jax.ShapeDtypeStruct((N, D), jnp.float32),
    ),
    mesh=_mesh,
    scratch_types=(
        [pltpu.VMEM((KBUF, CH_S), jnp.int32),
         pltpu.VMEM((KBUF, CH_S, D), jnp.float32),
         pltpu.VMEM_SHARED((N, D), jnp.float32)]
        + [pltpu.SemaphoreType.DMA] * (3 * KBUF)
    ),
)
def _sc_scatter(gnew_hbm, dst_hbm, zeros_hbm, out0_hbm, out1_hbm,
                idx_ring, rows, acc_sh, *sems):
    isem = sems[:KBUF]
    rsem = sems[KBUF:2 * KBUF]
    asem = sems[2 * KBUF:]
    cid = lax.axis_index("c")
    sid = lax.axis_index("s")

    r0 = sid * ROWS_PER_SUB
    pltpu.sync_copy(zeros_hbm.at[pl.ds(r0, ROWS_PER_SUB)],
                    acc_sh.at[pl.ds(r0, ROWS_PER_SUB)])

    @pl.when(sid == NS - 1)
    def _():
        t0 = NS * ROWS_PER_SUB
        pltpu.sync_copy(zeros_hbm.at[pl.ds(t0, ROWS_TAIL)],
                        acc_sh.at[pl.ds(t0, ROWS_TAIL)])

    plsc.subcore_barrier()

    base = (cid * NS + sid) * PER_W_S

    def group(gi, carry):
        rel = gi * (KBUF * CH_S)
        descs = []
        for b in range(KBUF):
            @pl.when(gi > 0)
            def _(b=b):
                pltpu.make_async_copy(
                    rows.at[b], acc_sh.at[idx_ring.at[b]], asem[b]).wait()

            off = base + rel + b * CH_S
            di = pltpu.async_copy(dst_hbm.at[pl.ds(off, CH_S)],
                                  idx_ring.at[b], isem[b])
            dr = pltpu.async_copy(gnew_hbm.at[pl.ds(off, CH_S)],
                                  rows.at[b], rsem[b])
            descs.append((di, dr))
        for b in range(KBUF):
            descs[b][0].wait()
            descs[b][1].wait()
            pltpu.async_copy(rows.at[b], acc_sh.at[idx_ring.at[b]],
                             asem[b], add=True)
        return carry

    lax.fori_loop(0, NGROUP_S, group, 0)
    for b in range(KBUF):
        pltpu.make_async_copy(
            rows.at[b], acc_sh.at[idx_ring.at[b]], asem[b]).wait()
    plsc.subcore_barrier()

    @pl.when(cid == 0)
    def _():
        pltpu.sync_copy(acc_sh.at[pl.ds(r0, ROWS_PER_SUB)],
                        out0_hbm.at[pl.ds(r0, ROWS_PER_SUB)])

        @pl.when(sid == NS - 1)
        def _():
            t0 = NS * ROWS_PER_SUB
            pltpu.sync_copy(acc_sh.at[pl.ds(t0, ROWS_TAIL)],
                            out0_hbm.at[pl.ds(t0, ROWS_TAIL)])

    @pl.when(cid == 1)
    def _():
        pltpu.sync_copy(acc_sh.at[pl.ds(r0, ROWS_PER_SUB)],
                        out1_hbm.at[pl.ds(r0, ROWS_PER_SUB)])

        @pl.when(sid == NS - 1)
        def _():
            t0 = NS * ROWS_PER_SUB
            pltpu.sync_copy(acc_sh.at[pl.ds(t0, ROWS_TAIL)],
                            out1_hbm.at[pl.ds(t0, ROWS_TAIL)])


def _m_body(h_ref, w2_ref, m_ref):
    m_ref[...] = jnp.dot(h_ref[...], w2_ref[...],
                         preferred_element_type=jnp.float32)


_tc_m = pl.pallas_call(
    _m_body,
    out_shape=jax.ShapeDtypeStruct((N, D), jnp.float32),
)

BLK = 1600


def _edge_body0(g_ref, ms_ref, w1_ref, be_ref, out_ref):
    acc = jnp.dot(g_ref[...], w1_ref[...], preferred_element_type=jnp.float32)
    out_ref[...] = jnp.maximum(acc + ms_ref[...] + be_ref[...], 0.0)


def _edge_bodyp(prev_ref, g_ref, ms_ref, w1_ref, be_ref, out_ref):
    del prev_ref
    acc = jnp.dot(g_ref[...], w1_ref[...], preferred_element_type=jnp.float32)
    out_ref[...] = jnp.maximum(acc + ms_ref[...] + be_ref[...], 0.0)


def _make_edge(piece):
    poff, ep, _ = PIECES[piece]
    boff = poff // BLK
    bpp = ep // BLK
    g_spec = pl.BlockSpec((BLK, D), lambda i: (boff + i, 0))
    ms_spec = pl.BlockSpec((BLK, D), lambda i: (i, 0))
    w1_spec = pl.BlockSpec((D, D), lambda i: (0, 0))
    be_spec = pl.BlockSpec((1, D), lambda i: (0, 0))
    out_spec = pl.BlockSpec((BLK, D), lambda i: (boff + i, 0))
    if piece == 0:
        return pl.pallas_call(
            _edge_body0,
            grid=(bpp,),
            in_specs=[g_spec, ms_spec, w1_spec, be_spec],
            out_specs=out_spec,
            out_shape=jax.ShapeDtypeStruct((E, D), jnp.float32),
        )
    return pl.pallas_call(
        _edge_bodyp,
        grid=(bpp,),
        in_specs=[pl.BlockSpec(memory_space=pl.ANY),
                  g_spec, ms_spec, w1_spec, be_spec],
        out_specs=out_spec,
        out_shape=jax.ShapeDtypeStruct((E, D), jnp.float32),
        input_output_aliases={0: 0},
    )


_tc_edges = [_make_edge(p) for p in range(P)]


def _node_body(h_ref, wv_ref, bv_ref, a0_ref, a1_ref, out_ref):
    acc = jnp.dot(h_ref[...], wv_ref[...], preferred_element_type=jnp.float32)
    out_ref[...] = jnp.maximum(acc + bv_ref[...] + a0_ref[...] + a1_ref[...],
                               0.0)


_tc_node = pl.pallas_call(
    _node_body,
    out_shape=jax.ShapeDtypeStruct((N, D), jnp.float32),
)


def kernel(h, g, edge_index, W_e, b_e, W_v, b_v):
    h2 = h[0]
    g2 = g[0]
    src = edge_index[0]
    dst = edge_index[1]
    W1 = W_e[:D]
    W2 = W_e[D:]
    zeros = jnp.zeros((N, D), jnp.float32)

    m = _tc_m(h2, W2)
    msrc = [_sc_gathers[p](m, src) for p in range(P)]
    gnew = _tc_edges[0](g2, msrc[0], W1, b_e.reshape(1, D))
    for p in range(1, P):
        gnew = _tc_edges[p](gnew, g2, msrc[p], W1, b_e.reshape(1, D))
    agg0, agg1 = _sc_scatter(gnew, dst, zeros)
    hnew = _tc_node(h2, W_v, b_v.reshape(1, D), agg0, agg1)
    return hnew[None], gnew[None]


# speedup vs baseline: 1.0694x; 1.0694x over previous
import functools

import jax
import jax.numpy as jnp
from jax import lax
from jax.experimental import pallas as pl
from jax.experimental.pallas import tpu as pltpu
from jax.experimental.pallas import tpu_sc as plsc

N = 10000
D = 128
E = 320000

NC = 2
NS = 16
NW = NC * NS

PIECES = [(0, 64000, 80), (64000, 128000, 80), (192000, 128000, 80)]
P = len(PIECES)
KBUF = 5

PER_W_S = E // NW
CH_S = 40
NCHUNK_S = PER_W_S // CH_S
NGROUP_S = NCHUNK_S // KBUF

ROWS_PER_SUB = 624
ROWS_TAIL = N - NS * ROWS_PER_SUB

_mesh = plsc.VectorSubcoreMesh(core_axis_name="c", subcore_axis_name="s")


def _make_gather(poff, ep, ch):
    per_w = ep // NW
    ngroup = per_w // (ch * KBUF)
    assert ngroup * ch * KBUF == per_w and ch % 8 == 0 and per_w % 8 == 0

    @functools.partial(
        pl.kernel,
        out_type=jax.ShapeDtypeStruct((ep, D), jnp.float32),
        mesh=_mesh,
        scratch_types=(
            [pltpu.VMEM((per_w,), jnp.int32),
             pltpu.VMEM((KBUF, ch, D), jnp.float32)]
            + [pltpu.SemaphoreType.DMA] * (2 * KBUF)
        ),
    )
    def _sc_gather(m_hbm, src_hbm, out_hbm, idx_all, rows, *sems):
        gsem = sems[:KBUF]
        wsem = sems[KBUF:]
        wid = lax.axis_index("s") * NC + lax.axis_index("c")
        base = wid * per_w
        pltpu.sync_copy(src_hbm.at[pl.ds(poff + base, per_w)], idx_all)

        def group(gi, carry):
            rel = gi * (KBUF * ch)
            descs = []
            for b in range(KBUF):
                @pl.when(gi > 0)
                def _(b=b):
                    pltpu.make_async_copy(
                        rows.at[b], out_hbm.at[pl.ds(base, ch)],
                        wsem[b]).wait()

                d = pltpu.async_copy(
                    m_hbm.at[idx_all.at[pl.ds(rel + b * ch, ch)]],
                    rows.at[b], gsem[b])
                descs.append(d)
            for b in range(KBUF):
                descs[b].wait()
                pltpu.async_copy(rows.at[b],
                                 out_hbm.at[pl.ds(base + rel + b * ch, ch)],
                                 wsem[b])
            return carry

        lax.fori_loop(0, ngroup, group, 0)
        for b in range(KBUF):
            pltpu.make_async_copy(
                rows.at[b], out_hbm.at[pl.ds(base, ch)], wsem[b]).wait()

    return _sc_gather


_sc_gathers = [_make_gather(*pc) for pc in PIECES]


@functools.partial(
    pl.kernel,
    out_type=(
        jax.ShapeDtypeStruct((N, D), jnp.float32),
        jax.ShapeDtypeStruct((N, D), jnp.float32),
    ),
    mesh=_mesh,
    scratch_types=(
        [pltpu.VMEM((KBUF, CH_S), jnp.int32),
         pltpu.VMEM((KBUF, CH_S, D), jnp.float32),
         pltpu.VMEM_SHARED((N, D), jnp.float32)]
        + [pltpu.SemaphoreType.DMA] * (3 * KBUF)
    ),
)
def _sc_scatter(gnew_hbm, dst_hbm, zeros_hbm, out0_hbm, out1_hbm,
                idx_ring, rows, acc_sh, *sems):
    isem = sems[:KBUF]
    rsem = sems[KBUF:2 * KBUF]
    asem = sems[2 * KBUF:]
    cid = lax.axis_index("c")
    sid = lax.axis_index("s")

    r0 = sid * ROWS_PER_SUB
    pltpu.sync_copy(zeros_hbm.at[pl.ds(r0, ROWS_PER_SUB)],
                    acc_sh.at[pl.ds(r0, ROWS_PER_SUB)])

    @pl.when(sid == NS - 1)
    def _():
        t0 = NS * ROWS_PER_SUB
        pltpu.sync_copy(zeros_hbm.at[pl.ds(t0, ROWS_TAIL)],
                        acc_sh.at[pl.ds(t0, ROWS_TAIL)])

    plsc.subcore_barrier()

    base = (cid * NS + sid) * PER_W_S

    def group(gi, carry):
        rel = gi * (KBUF * CH_S)
        descs = []
        for b in range(KBUF):
            @pl.when(gi > 0)
            def _(b=b):
                pltpu.make_async_copy(
                    rows.at[b], acc_sh.at[idx_ring.at[b]], asem[b]).wait()

            off = base + rel + b * CH_S
            di = pltpu.async_copy(dst_hbm.at[pl.ds(off, CH_S)],
                                  idx_ring.at[b], isem[b])
            dr = pltpu.async_copy(gnew_hbm.at[pl.ds(off, CH_S)],
                                  rows.at[b], rsem[b])
            descs.append((di, dr))
        for b in range(KBUF):
            descs[b][0].wait()
            descs[b][1].wait()
            pltpu.async_copy(rows.at[b], acc_sh.at[idx_ring.at[b]],
                             asem[b], add=True)
        return carry

    lax.fori_loop(0, NGROUP_S, group, 0)
    for b in range(KBUF):
        pltpu.make_async_copy(
            rows.at[b], acc_sh.at[idx_ring.at[b]], asem[b]).wait()
    plsc.subcore_barrier()

    @pl.when(cid == 0)
    def _():
        pltpu.sync_copy(acc_sh.at[pl.ds(r0, ROWS_PER_SUB)],
                        out0_hbm.at[pl.ds(r0, ROWS_PER_SUB)])

        @pl.when(sid == NS - 1)
        def _():
            t0 = NS * ROWS_PER_SUB
            pltpu.sync_copy(acc_sh.at[pl.ds(t0, ROWS_TAIL)],
                            out0_hbm.at[pl.ds(t0, ROWS_TAIL)])

    @pl.when(cid == 1)
    def _():
        pltpu.sync_copy(acc_sh.at[pl.ds(r0, ROWS_PER_SUB)],
                        out1_hbm.at[pl.ds(r0, ROWS_PER_SUB)])

        @pl.when(sid == NS - 1)
        def _():
            t0 = NS * ROWS_PER_SUB
            pltpu.sync_copy(acc_sh.at[pl.ds(t0, ROWS_TAIL)],
                            out1_hbm.at[pl.ds(t0, ROWS_TAIL)])


def _m_body(h_ref, w2_ref, m_ref):
    m_ref[...] = jnp.dot(h_ref[...], w2_ref[...],
                         preferred_element_type=jnp.float32)


_tc_m = pl.pallas_call(
    _m_body,
    out_shape=jax.ShapeDtypeStruct((N, D), jnp.float32),
)

BLK = 2560


def _edge_body0(g_ref, ms_ref, w1_ref, be_ref, out_ref):
    acc = jnp.dot(g_ref[...], w1_ref[...], preferred_element_type=jnp.float32)
    out_ref[...] = jnp.maximum(acc + ms_ref[...] + be_ref[...], 0.0)


def _edge_bodyp(prev_ref, g_ref, ms_ref, w1_ref, be_ref, out_ref):
    del prev_ref
    acc = jnp.dot(g_ref[...], w1_ref[...], preferred_element_type=jnp.float32)
    out_ref[...] = jnp.maximum(acc + ms_ref[...] + be_ref[...], 0.0)


def _make_edge(piece):
    poff, ep, _ = PIECES[piece]
    boff = poff // BLK
    bpp = ep // BLK
    g_spec = pl.BlockSpec((BLK, D), lambda i: (boff + i, 0))
    ms_spec = pl.BlockSpec((BLK, D), lambda i: (i, 0))
    w1_spec = pl.BlockSpec((D, D), lambda i: (0, 0))
    be_spec = pl.BlockSpec((1, D), lambda i: (0, 0))
    out_spec = pl.BlockSpec((BLK, D), lambda i: (boff + i, 0))
    if piece == 0:
        return pl.pallas_call(
            _edge_body0,
            grid=(bpp,),
            in_specs=[g_spec, ms_spec, w1_spec, be_spec],
            out_specs=out_spec,
            out_shape=jax.ShapeDtypeStruct((E, D), jnp.float32),
        )
    return pl.pallas_call(
        _edge_bodyp,
        grid=(bpp,),
        in_specs=[pl.BlockSpec(memory_space=pl.ANY),
                  g_spec, ms_spec, w1_spec, be_spec],
        out_specs=out_spec,
        out_shape=jax.ShapeDtypeStruct((E, D), jnp.float32),
        input_output_aliases={0: 0},
    )


_tc_edges = [_make_edge(p) for p in range(P)]


def _node_body(h_ref, wv_ref, bv_ref, a0_ref, a1_ref, out_ref):
    acc = jnp.dot(h_ref[...], wv_ref[...], preferred_element_type=jnp.float32)
    out_ref[...] = jnp.maximum(acc + bv_ref[...] + a0_ref[...] + a1_ref[...],
                               0.0)


_tc_node = pl.pallas_call(
    _node_body,
    out_shape=jax.ShapeDtypeStruct((N, D), jnp.float32),
)


def kernel(h, g, edge_index, W_e, b_e, W_v, b_v):
    h2 = h[0]
    g2 = g[0]
    src = edge_index[0]
    dst = edge_index[1]
    W1 = W_e[:D]
    W2 = W_e[D:]
    zeros = jnp.zeros((N, D), jnp.float32)

    m = _tc_m(h2, W2)
    msrc = [_sc_gathers[p](m, src) for p in range(P)]
    gnew = _tc_edges[0](g2, msrc[0], W1, b_e.reshape(1, D))
    for p in range(1, P):
        gnew = _tc_edges[p](gnew, g2, msrc[p], W1, b_e.reshape(1, D))
    agg0, agg1 = _sc_scatter(gnew, dst, zeros)
    hnew = _tc_node(h2, W_v, b_v.reshape(1, D), agg0, agg1)
    return hnew[None], gnew[None]


# speedup vs baseline: 1.0800x; 1.0099x over previous
import functools

import jax
import jax.numpy as jnp
from jax import lax
from jax.experimental import pallas as pl
from jax.experimental.pallas import tpu as pltpu
from jax.experimental.pallas import tpu_sc as plsc

N = 10000
D = 128
E = 320000

NC = 2
NS = 16
NW = NC * NS

PIECES = [(0, 64000, 80), (64000, 128000, 80), (192000, 128000, 80)]
P = len(PIECES)
KBUF = 5

PER_W_S = E // NW
CH_S = 40
NCHUNK_S = PER_W_S // CH_S
NGROUP_S = NCHUNK_S // KBUF

ROWS_PER_SUB = 624
ROWS_TAIL = N - NS * ROWS_PER_SUB

_mesh = plsc.VectorSubcoreMesh(core_axis_name="c", subcore_axis_name="s")


def _make_gather(poff, ep, ch):
    per_w = ep // NW
    ngroup = per_w // (ch * KBUF)
    assert ngroup * ch * KBUF == per_w and ch % 8 == 0 and per_w % 8 == 0

    @functools.partial(
        pl.kernel,
        out_type=jax.ShapeDtypeStruct((ep, D), jnp.float32),
        mesh=_mesh,
        scratch_types=(
            [pltpu.VMEM((per_w,), jnp.int32),
             pltpu.VMEM((KBUF, ch, D), jnp.float32)]
            + [pltpu.SemaphoreType.DMA] * (2 * KBUF)
        ),
    )
    def _sc_gather(m_hbm, src_hbm, out_hbm, idx_all, rows, *sems):
        gsem = sems[:KBUF]
        wsem = sems[KBUF:]
        wid = lax.axis_index("s") * NC + lax.axis_index("c")
        base = wid * per_w
        pltpu.sync_copy(src_hbm.at[pl.ds(poff + base, per_w)], idx_all)

        def group(gi, carry):
            rel = gi * (KBUF * ch)
            descs = []
            for b in range(KBUF):
                @pl.when(gi > 0)
                def _(b=b):
                    pltpu.make_async_copy(
                        rows.at[b], out_hbm.at[pl.ds(base, ch)],
                        wsem[b]).wait()

                d = pltpu.async_copy(
                    m_hbm.at[idx_all.at[pl.ds(rel + b * ch, ch)]],
                    rows.at[b], gsem[b])
                descs.append(d)
            for b in range(KBUF):
                descs[b].wait()
                pltpu.async_copy(rows.at[b],
                                 out_hbm.at[pl.ds(base + rel + b * ch, ch)],
                                 wsem[b])
            return carry

        lax.fori_loop(0, ngroup, group, 0)
        for b in range(KBUF):
            pltpu.make_async_copy(
                rows.at[b], out_hbm.at[pl.ds(base, ch)], wsem[b]).wait()

    return _sc_gather


_sc_gathers = [_make_gather(*pc) for pc in PIECES]


@functools.partial(
    pl.kernel,
    out_type=(
        jax.ShapeDtypeStruct((N, D), jnp.float32),
        jax.ShapeDtypeStruct((N, D), jnp.float32),
    ),
    mesh=_mesh,
    scratch_types=(
        [pltpu.VMEM((KBUF, CH_S), jnp.int32),
         pltpu.VMEM((KBUF, CH_S, D), jnp.float32),
         pltpu.VMEM_SHARED((N, D), jnp.float32)]
        + [pltpu.SemaphoreType.DMA] * (3 * KBUF)
    ),
)
def _sc_scatter(gnew_hbm, dst_hbm, zeros_hbm, out0_hbm, out1_hbm,
                idx_ring, rows, acc_sh, *sems):
    isem = sems[:KBUF]
    rsem = sems[KBUF:2 * KBUF]
    asem = sems[2 * KBUF:]
    cid = lax.axis_index("c")
    sid = lax.axis_index("s")

    r0 = sid * ROWS_PER_SUB
    pltpu.sync_copy(zeros_hbm.at[pl.ds(r0, ROWS_PER_SUB)],
                    acc_sh.at[pl.ds(r0, ROWS_PER_SUB)])

    @pl.when(sid == NS - 1)
    def _():
        t0 = NS * ROWS_PER_SUB
        pltpu.sync_copy(zeros_hbm.at[pl.ds(t0, ROWS_TAIL)],
                        acc_sh.at[pl.ds(t0, ROWS_TAIL)])

    plsc.subcore_barrier()

    base = (cid * NS + sid) * PER_W_S

    def group(gi, carry):
        rel = gi * (KBUF * CH_S)
        descs = []
        for b in range(KBUF):
            @pl.when(gi > 0)
            def _(b=b):
                pltpu.make_async_copy(
                    rows.at[b], acc_sh.at[idx_ring.at[b]], asem[b]).wait()

            off = base + rel + b * CH_S
            di = pltpu.async_copy(dst_hbm.at[pl.ds(off, CH_S)],
                                  idx_ring.at[b], isem[b])
            dr = pltpu.async_copy(gnew_hbm.at[pl.ds(off, CH_S)],
                                  rows.at[b], rsem[b])
            descs.append((di, dr))
        for b in range(KBUF):
            descs[b][0].wait()
            descs[b][1].wait()
            pltpu.async_copy(rows.at[b], acc_sh.at[idx_ring.at[b]],
                             asem[b], add=True)
        return carry

    lax.fori_loop(0, NGROUP_S, group, 0)
    for b in range(KBUF):
        pltpu.make_async_copy(
            rows.at[b], acc_sh.at[idx_ring.at[b]], asem[b]).wait()
    plsc.subcore_barrier()

    @pl.when(cid == 0)
    def _():
        pltpu.sync_copy(acc_sh.at[pl.ds(r0, ROWS_PER_SUB)],
                        out0_hbm.at[pl.ds(r0, ROWS_PER_SUB)])

        @pl.when(sid == NS - 1)
        def _():
            t0 = NS * ROWS_PER_SUB
            pltpu.sync_copy(acc_sh.at[pl.ds(t0, ROWS_TAIL)],
                            out0_hbm.at[pl.ds(t0, ROWS_TAIL)])

    @pl.when(cid == 1)
    def _():
        pltpu.sync_copy(acc_sh.at[pl.ds(r0, ROWS_PER_SUB)],
                        out1_hbm.at[pl.ds(r0, ROWS_PER_SUB)])

        @pl.when(sid == NS - 1)
        def _():
            t0 = NS * ROWS_PER_SUB
            pltpu.sync_copy(acc_sh.at[pl.ds(t0, ROWS_TAIL)],
                            out1_hbm.at[pl.ds(t0, ROWS_TAIL)])


def _m_body(h_ref, w2_ref, m_ref):
    m_ref[...] = jnp.dot(h_ref[...], w2_ref[...],
                         preferred_element_type=jnp.float32)


_tc_m = pl.pallas_call(
    _m_body,
    out_shape=jax.ShapeDtypeStruct((N, D), jnp.float32),
)

BLK = 2560


def _edge_body0(g_ref, hs_ref, w1_ref, w2_ref, be_ref, out_ref):
    acc = jnp.dot(g_ref[...], w1_ref[...], preferred_element_type=jnp.float32)
    acc += jnp.dot(hs_ref[...], w2_ref[...],
                   preferred_element_type=jnp.float32)
    out_ref[...] = jnp.maximum(acc + be_ref[...], 0.0)


def _edge_bodyp(prev_ref, g_ref, ms_ref, w1_ref, be_ref, out_ref):
    del prev_ref
    acc = jnp.dot(g_ref[...], w1_ref[...], preferred_element_type=jnp.float32)
    out_ref[...] = jnp.maximum(acc + ms_ref[...] + be_ref[...], 0.0)


def _make_edge(piece):
    poff, ep, _ = PIECES[piece]
    boff = poff // BLK
    bpp = ep // BLK
    g_spec = pl.BlockSpec((BLK, D), lambda i: (boff + i, 0))
    ms_spec = pl.BlockSpec((BLK, D), lambda i: (i, 0))
    w1_spec = pl.BlockSpec((D, D), lambda i: (0, 0))
    be_spec = pl.BlockSpec((1, D), lambda i: (0, 0))
    out_spec = pl.BlockSpec((BLK, D), lambda i: (boff + i, 0))
    if piece == 0:
        return pl.pallas_call(
            _edge_body0,
            grid=(bpp,),
            in_specs=[g_spec, ms_spec, w1_spec, w1_spec, be_spec],
            out_specs=out_spec,
            out_shape=jax.ShapeDtypeStruct((E, D), jnp.float32),
        )
    return pl.pallas_call(
        _edge_bodyp,
        grid=(bpp,),
        in_specs=[pl.BlockSpec(memory_space=pl.ANY),
                  g_spec, ms_spec, w1_spec, be_spec],
        out_specs=out_spec,
        out_shape=jax.ShapeDtypeStruct((E, D), jnp.float32),
        input_output_aliases={0: 0},
    )


_tc_edges = [_make_edge(p) for p in range(P)]


def _node_body(h_ref, wv_ref, bv_ref, a0_ref, a1_ref, out_ref):
    acc = jnp.dot(h_ref[...], wv_ref[...], preferred_element_type=jnp.float32)
    out_ref[...] = jnp.maximum(acc + bv_ref[...] + a0_ref[...] + a1_ref[...],
                               0.0)


_tc_node = pl.pallas_call(
    _node_body,
    out_shape=jax.ShapeDtypeStruct((N, D), jnp.float32),
)


def kernel(h, g, edge_index, W_e, b_e, W_v, b_v):
    h2 = h[0]
    g2 = g[0]
    src = edge_index[0]
    dst = edge_index[1]
    W1 = W_e[:D]
    W2 = W_e[D:]
    zeros = jnp.zeros((N, D), jnp.float32)

    hs0 = _sc_gathers[0](h2, src)
    m = _tc_m(h2, W2)
    msrc = [_sc_gathers[p](m, src) for p in range(1, P)]
    gnew = _tc_edges[0](g2, hs0, W1, W2, b_e.reshape(1, D))
    for p in range(1, P):
        gnew = _tc_edges[p](gnew, g2, msrc[p - 1], W1, b_e.reshape(1, D))
    agg0, agg1 = _sc_scatter(gnew, dst, zeros)
    hnew = _tc_node(h2, W_v, b_v.reshape(1, D), agg0, agg1)
    return hnew[None], gnew[None]
